# baseline (speedup 1.0000x reference)
import numpy as np

import jax
import jax.numpy as jnp
from jax import lax
from jax.experimental import pallas as pl
from jax.experimental.pallas import tpu as pltpu

_B = 512
_EPS = 1e-5


def _band1(w_in):
    d = np.zeros((3, w_in, w_in), np.float32)
    for dx in range(3):
        for xo in range(w_in):
            xi = xo + dx - 1
            if 0 <= xi < w_in:
                d[dx, xi, xo] = 1.0
    return d


def _band2(w_in):
    w_out = w_in // 2
    d = np.zeros((3, w_in, w_out), np.float32)
    for dx in range(3):
        for xo in range(w_out):
            xi = 2 * xo + dx
            if xi < w_in:
                d[dx, xi, xo] = 1.0
    return d


def _pool_mat(w_in, c, k, scale):
    p = np.zeros((w_in * c, (w_in // k) * c), np.float32)
    for x in range(w_in):
        for ch in range(c):
            p[x * c + ch, (x // k) * c + ch] = scale
    return p


def _chan_fold(w_in, c):
    r = np.zeros((w_in * c, c), np.float32)
    for x in range(w_in):
        for ch in range(c):
            r[x * c + ch, ch] = 1.0
    return r


_D1_32 = _band1(32)
_D2_32 = _band2(32)
_D2_16 = _band2(16)
_P0 = _pool_mat(32, 16, 4, 1.0 / 16.0)
_P1 = _pool_mat(16, 32, 4, 1.0 / 16.0)
_PH = _chan_fold(8, 64) / 64.0
_R16 = _chan_fold(32, 16)
_RT16 = _chan_fold(32, 16).T


def _rowshift(v, s):
    bs, h, l = v.shape
    z = jnp.zeros((bs, 1, l), jnp.float32)
    if s == -1:
        return jnp.concatenate([z, v[:, :h - 1]], axis=1)
    if s == 1:
        return jnp.concatenate([v[:, 1:], z], axis=1)
    return v


def _im_s1(v):
    bs, h, l = v.shape
    im = jnp.concatenate([_rowshift(v, dy - 1) for dy in range(3)], axis=-1)
    return im.reshape(bs * h, 3 * l)


def _im_s2(v):
    bs, h2, l = v.shape
    h = h2 // 2
    par = v.reshape(bs, h, 2, l)
    ev = par[:, :, 0]
    od = par[:, :, 1]
    z = jnp.zeros((bs, 1, l), jnp.float32)
    ev1 = jnp.concatenate([ev[:, 1:], z], axis=1)
    im = jnp.concatenate([ev, od, ev1], axis=-1)
    return im.reshape(bs * h, 3 * l)


def _mm(a, b):
    return jnp.dot(a, b, preferred_element_type=jnp.float32,
                   precision=lax.Precision.HIGHEST)


def _stats_body(x_ref, wbs_ref, r16_ref, stats_ref):
    bs = x_ref.shape[0]
    y = _mm(_im_s1(x_ref[...]), wbs_ref[...])
    r16 = r16_ref[...]
    s = _mm(jnp.sum(y, axis=0)[None, :], r16)
    s2 = _mm(jnp.sum(y * y, axis=0)[None, :], r16)
    rows = lax.broadcasted_iota(jnp.int32, (8, 16), 0)
    upd = jnp.where(rows == 0, s, jnp.where(rows == 1, s2, 0.0))
    prev = jnp.where(pl.program_id(0) == 0, 0.0, stats_ref[...])
    stats_ref[...] = prev + upd


def _stats(xw, wbs, bs):
    return pl.pallas_call(
        _stats_body,
        grid=(_B // bs,),
        in_specs=[
            pl.BlockSpec((bs, 32, 96), lambda i: (i, 0, 0)),
            pl.BlockSpec((288, 512), lambda i: (0, 0)),
            pl.BlockSpec((512, 16), lambda i: (0, 0)),
        ],
        out_specs=pl.BlockSpec((8, 16), lambda i: (0, 0)),
        out_shape=jax.ShapeDtypeStruct((8, 16), jnp.float32),
        compiler_params=pltpu.CompilerParams(
            dimension_semantics=("arbitrary",)),
    )(xw, wbs, jnp.asarray(_R16))


def _net_body(x_ref, stats_ref, sc_ref, bi_ref, wbs_ref, wb0_ref, wb1_ref,
              wb2_ref, r0w1_ref, r0b1_ref, r0w2_ref, r0b2_ref,
              r1w1_ref, r1b1_ref, r1w2_ref, r1b2_ref, fcw_ref, fcb_ref,
              rt16_ref, p0_ref, p1_ref, ph_ref,
              out_ref):
    bs = x_ref.shape[0]
    y = _mm(_im_s1(x_ref[...]), wbs_ref[...])
    n = jnp.float32(_B * 1024)
    st = stats_ref[...]
    mean = st[0:1, :] / n
    var = st[1:2, :] / n - mean * mean
    inv = sc_ref[...] * lax.rsqrt(var + _EPS)
    sh = bi_ref[...] - mean * inv
    rt16 = rt16_ref[...]
    inv512 = _mm(inv, rt16)
    sh512 = _mm(sh, rt16)
    xn = jnp.maximum(y * inv512 + sh512, 0.0).reshape(bs, 32, 512)
    b0 = jnp.maximum(_mm(_im_s1(xn), wb0_ref[...]), 0.0)
    b0 = b0.reshape(bs, 32, 512)
    t = b0.reshape(bs, 8, 4, 512)
    ys = t[:, :, 0] + t[:, :, 1] + t[:, :, 2] + t[:, :, 3]
    pool0 = _mm(ys.reshape(bs * 8, 512), p0_ref[...])
    pool0 = pool0.reshape(bs, 8, 128).reshape(bs, 1024)
    h0 = jnp.maximum(_mm(pool0, r0w1_ref[...]) + r0b1_ref[...], 0.0)
    lg0 = _mm(h0, r0w2_ref[...]) + r0b2_ref[...]
    m0 = (lg0[:, 1:2] > lg0[:, 0:1])[:, :, None]
    r1 = _mm(_im_s2(b0), wb1_ref[...]).reshape(bs, 16, 1024)
    o1 = jnp.maximum(jnp.where(m0, r1[:, :, 512:], r1[:, :, :512]), 0.0)
    t1 = o1.reshape(bs, 4, 4, 512)
    ys1 = t1[:, :, 0] + t1[:, :, 1] + t1[:, :, 2] + t1[:, :, 3]
    pool1 = _mm(ys1.reshape(bs * 4, 512), p1_ref[...])
    pool1 = pool1.reshape(bs, 4, 128).reshape(bs, 512)
    h1 = jnp.maximum(_mm(pool1, r1w1_ref[...]) + r1b1_ref[...], 0.0)
    lg1 = _mm(h1, r1w2_ref[...]) + r1b2_ref[...]
    mx = jnp.max(lg1, axis=1, keepdims=True)
    colid = lax.broadcasted_iota(jnp.int32, lg1.shape, 1)
    idx1 = jnp.min(jnp.where(lg1 == mx, colid, 4), axis=1,
                   keepdims=True)
    r2 = _mm(_im_s2(o1), wb2_ref[...]).reshape(bs, 8, 2048)
    acc = jnp.zeros((bs, 8, 512), jnp.float32)
    for e in range(4):
        me = (idx1 == e).astype(jnp.float32)[:, :, None]
        acc = acc + me * r2[:, :, 512 * e:512 * e + 512]
    o2 = jnp.maximum(acc, 0.0)
    feat = _mm(jnp.sum(o2, axis=1), ph_ref[...])
    out_ref[...] = _mm(feat, fcw_ref[...]) + fcb_ref[...]


def _net(xw, stats, bnsc, bnbi, wbs, wb0, wb1, wb2,
         r0w1, r0b1, r0w2, r0b2, r1w1, r1b1, r1w2, r1b2, fcw, fcb, bs):
    return pl.pallas_call(
        _net_body,
        grid=(_B // bs,),
        in_specs=[
            pl.BlockSpec((bs, 32, 96), lambda i: (i, 0, 0)),
            pl.BlockSpec((8, 16), lambda i: (0, 0)),
            pl.BlockSpec((1, 16), lambda i: (0, 0)),
            pl.BlockSpec((1, 16), lambda i: (0, 0)),
            pl.BlockSpec((288, 512), lambda i: (0, 0)),
            pl.BlockSpec((1536, 512), lambda i: (0, 0)),
            pl.BlockSpec((1536, 1024), lambda i: (0, 0)),
            pl.BlockSpec((1536, 2048), lambda i: (0, 0)),
            pl.BlockSpec((1024, 128), lambda i: (0, 0)),
            pl.BlockSpec((1, 128), lambda i: (0, 0)),
            pl.BlockSpec((128, 2), lambda i: (0, 0)),
            pl.BlockSpec((1, 2), lambda i: (0, 0)),
            pl.BlockSpec((512, 128), lambda i: (0, 0)),
            pl.BlockSpec((1, 128), lambda i: (0, 0)),
            pl.BlockSpec((128, 4), lambda i: (0, 0)),
            pl.BlockSpec((1, 4), lambda i: (0, 0)),
            pl.BlockSpec((64, 10), lambda i: (0, 0)),
            pl.BlockSpec((1, 10), lambda i: (0, 0)),
            pl.BlockSpec((16, 512), lambda i: (0, 0)),
            pl.BlockSpec((512, 128), lambda i: (0, 0)),
            pl.BlockSpec((512, 128), lambda i: (0, 0)),
            pl.BlockSpec((512, 64), lambda i: (0, 0)),
        ],
        out_specs=pl.BlockSpec((bs, 10), lambda i: (i, 0)),
        out_shape=jax.ShapeDtypeStruct((_B, 10), jnp.float32),
    )(xw, stats, bnsc, bnbi, wbs, wb0, wb1, wb2,
      r0w1, r0b1, r0w2, r0b2, r1w1, r1b1, r1w2, r1b2, fcw, fcb,
      jnp.asarray(_RT16), jnp.asarray(_P0), jnp.asarray(_P1), jnp.asarray(_PH))


def kernel(x, labels, temperature, conv1_w, bn1_scale, bn1_bias, block0_w,
           block1_ws, block2_ws, r0_w1, r0_b1, r0_w2, r0_b2,
           r1_w1, r1_b1, r1_w2, r1_b2, fc_w, fc_b):
    xw = jnp.transpose(x, (0, 2, 1, 3)).reshape(_B, 32, 96)

    d1 = jnp.asarray(_D1_32)
    d2 = jnp.asarray(_D2_32)
    d2b = jnp.asarray(_D2_16)
    w1t = jnp.transpose(conv1_w, (2, 3, 1, 0))
    wbs = jnp.einsum('axo,yacp->ycxop', d1, w1t, precision=lax.Precision.HIGHEST).reshape(288, 512)
    w0t = jnp.transpose(block0_w, (2, 3, 1, 0))
    wb0 = jnp.einsum('axo,yacp->yxcop', d1, w0t, precision=lax.Precision.HIGHEST).reshape(1536, 512)
    w1e = jnp.transpose(block1_ws, (3, 4, 2, 0, 1))
    wb1 = jnp.einsum('axo,yacep->yxceop', d2, w1e, precision=lax.Precision.HIGHEST).reshape(1536, 1024)
    w2e = jnp.transpose(block2_ws, (3, 4, 2, 0, 1))
    wb2 = jnp.einsum('axo,yacep->yxceop', d2b, w2e, precision=lax.Precision.HIGHEST).reshape(1536, 2048)
    r0w1p = jnp.transpose(r0_w1.reshape(16, 8, 8, 128),
                          (1, 2, 0, 3)).reshape(1024, 128)
    r1w1p = jnp.transpose(r1_w1.reshape(32, 4, 4, 128),
                          (1, 2, 0, 3)).reshape(512, 128)

    stats = _stats(xw, wbs, bs=32)
    logits = _net(xw, stats, bn1_scale.reshape(1, 16), bn1_bias.reshape(1, 16),
                  wbs, wb0, wb1, wb2,
                  r0w1p, r0_b1.reshape(1, 128), r0_w2, r0_b2.reshape(1, 2),
                  r1w1p, r1_b1.reshape(1, 128), r1_w2, r1_b2.reshape(1, 4),
                  fc_w, fc_b.reshape(1, 10), bs=32)
    return logits


# speedup vs baseline: 2.2515x; 2.2515x over previous
import numpy as np

import jax
import jax.numpy as jnp
from jax import lax
from jax.experimental import pallas as pl
from jax.experimental.pallas import tpu as pltpu

_B = 512
_EPS = 1e-5


def _band1(w_in):
    d = np.zeros((3, w_in, w_in), np.float32)
    for dx in range(3):
        for xo in range(w_in):
            xi = xo + dx - 1
            if 0 <= xi < w_in:
                d[dx, xi, xo] = 1.0
    return d


def _band2(w_in):
    w_out = w_in // 2
    d = np.zeros((3, w_in, w_out), np.float32)
    for dx in range(3):
        for xo in range(w_out):
            xi = 2 * xo + dx
            if xi < w_in:
                d[dx, xi, xo] = 1.0
    return d


def _pool_mat(w_in, c, k, scale):
    p = np.zeros((w_in * c, (w_in // k) * c), np.float32)
    for x in range(w_in):
        for ch in range(c):
            p[x * c + ch, (x // k) * c + ch] = scale
    return p


def _chan_fold(w_in, c):
    r = np.zeros((w_in * c, c), np.float32)
    for x in range(w_in):
        for ch in range(c):
            r[x * c + ch, ch] = 1.0
    return r


_D1_32 = _band1(32)
_D2_32 = _band2(32)
_D2_16 = _band2(16)
_P0 = _pool_mat(32, 16, 4, 1.0 / 16.0)
_P1 = _pool_mat(16, 32, 4, 1.0 / 16.0)
_PH = _chan_fold(8, 64) / 64.0
_R16 = _chan_fold(32, 16)
_RT16 = _chan_fold(32, 16).T


def _rowshift(v, s):
    bs, h, l = v.shape
    z = jnp.zeros((bs, 1, l), jnp.float32)
    if s == -1:
        return jnp.concatenate([z, v[:, :h - 1]], axis=1)
    if s == 1:
        return jnp.concatenate([v[:, 1:], z], axis=1)
    return v


def _im_s1(v):
    bs, h, l = v.shape
    im = jnp.concatenate([_rowshift(v, dy - 1) for dy in range(3)], axis=-1)
    return im.reshape(bs * h, 3 * l)


def _im_stem(x_ref):
    bs = x_ref.shape[0]
    vt = jnp.transpose(x_ref[...], (0, 2, 1, 3))
    pieces = []
    for dy in range(3):
        for ci in range(3):
            pieces.append(_rowshift(vt[:, :, ci, :], dy - 1))
    return jnp.concatenate(pieces, axis=-1).reshape(bs * 32, 288)


def _im_s2(v):
    bs, h2, l = v.shape
    h = h2 // 2
    par = v.reshape(bs, h, 2, l)
    ev = par[:, :, 0]
    od = par[:, :, 1]
    z = jnp.zeros((bs, 1, l), jnp.float32)
    ev1 = jnp.concatenate([ev[:, 1:], z], axis=1)
    im = jnp.concatenate([ev, od, ev1], axis=-1)
    return im.reshape(bs * h, 3 * l)


def _mm(a, b):
    return jnp.dot(a, b, preferred_element_type=jnp.float32)


def _mmx(a, b):
    return jnp.dot(a, b, preferred_element_type=jnp.float32,
                   precision=lax.Precision.HIGHEST)


def _stats_body(x_ref, wbs_ref, r16_ref, stats_ref):
    bs = x_ref.shape[0]
    y = _mm(_im_stem(x_ref), wbs_ref[...])
    r16 = r16_ref[...]
    s = _mm(jnp.sum(y, axis=0)[None, :], r16)
    s2 = _mm(jnp.sum(y * y, axis=0)[None, :], r16)
    rows = lax.broadcasted_iota(jnp.int32, (8, 16), 0)
    upd = jnp.where(rows == 0, s, jnp.where(rows == 1, s2, 0.0))
    prev = jnp.where(pl.program_id(0) == 0, 0.0, stats_ref[...])
    stats_ref[...] = prev + upd


def _stats(xw, wbs, bs):
    return pl.pallas_call(
        _stats_body,
        grid=(_B // bs,),
        in_specs=[
            pl.BlockSpec((bs, 3, 32, 32), lambda i: (i, 0, 0, 0)),
            pl.BlockSpec((288, 512), lambda i: (0, 0)),
            pl.BlockSpec((512, 16), lambda i: (0, 0)),
        ],
        out_specs=pl.BlockSpec((8, 16), lambda i: (0, 0)),
        out_shape=jax.ShapeDtypeStruct((8, 16), jnp.float32),
        compiler_params=pltpu.CompilerParams(
            dimension_semantics=("arbitrary",)),
    )(xw, wbs, jnp.asarray(_R16))


def _net_body(x_ref, stats_ref, sc_ref, bi_ref, wbs_ref, wb0_ref, wb1_ref,
              wb2_ref, r0w1_ref, r0b1_ref, r0w2_ref, r0b2_ref,
              r1w1_ref, r1b1_ref, r1w2_ref, r1b2_ref, fcw_ref, fcb_ref,
              rt16_ref, p0_ref, p1_ref, ph_ref,
              out_ref):
    bs = x_ref.shape[0]
    y = _mm(_im_stem(x_ref), wbs_ref[...])
    n = jnp.float32(_B * 1024)
    st = stats_ref[...]
    mean = st[0:1, :] / n
    var = st[1:2, :] / n - mean * mean
    inv = sc_ref[...] * lax.rsqrt(var + _EPS)
    sh = bi_ref[...] - mean * inv
    rt16 = rt16_ref[...]
    inv512 = _mm(inv, rt16)
    sh512 = _mm(sh, rt16)
    xn = jnp.maximum(y * inv512 + sh512, 0.0).reshape(bs, 32, 512)
    b0 = jnp.maximum(_mm(_im_s1(xn), wb0_ref[...]), 0.0)
    b0 = b0.reshape(bs, 32, 512)
    t = b0.reshape(bs, 8, 4, 512)
    ys = t[:, :, 0] + t[:, :, 1] + t[:, :, 2] + t[:, :, 3]
    pool0 = _mmx(ys.reshape(bs * 8, 512), p0_ref[...])
    pool0 = pool0.reshape(bs, 8, 128).reshape(bs, 1024)
    h0 = jnp.maximum(_mmx(pool0, r0w1_ref[...]) + r0b1_ref[...], 0.0)
    lg0 = _mmx(h0, r0w2_ref[...]) + r0b2_ref[...]
    m0 = (lg0[:, 1:2] > lg0[:, 0:1])[:, :, None]
    r1 = _mm(_im_s2(b0), wb1_ref[...]).reshape(bs, 16, 1024)
    o1 = jnp.maximum(jnp.where(m0, r1[:, :, 512:], r1[:, :, :512]), 0.0)
    t1 = o1.reshape(bs, 4, 4, 512)
    ys1 = t1[:, :, 0] + t1[:, :, 1] + t1[:, :, 2] + t1[:, :, 3]
    pool1 = _mmx(ys1.reshape(bs * 4, 512), p1_ref[...])
    pool1 = pool1.reshape(bs, 4, 128).reshape(bs, 512)
    h1 = jnp.maximum(_mmx(pool1, r1w1_ref[...]) + r1b1_ref[...], 0.0)
    lg1 = _mmx(h1, r1w2_ref[...]) + r1b2_ref[...]
    mx = jnp.max(lg1, axis=1, keepdims=True)
    colid = lax.broadcasted_iota(jnp.int32, lg1.shape, 1)
    idx1 = jnp.min(jnp.where(lg1 == mx, colid, 4), axis=1,
                   keepdims=True)
    r2 = _mm(_im_s2(o1), wb2_ref[...]).reshape(bs, 8, 2048)
    acc = jnp.zeros((bs, 8, 512), jnp.float32)
    for e in range(4):
        me = (idx1 == e).astype(jnp.float32)[:, :, None]
        acc = acc + me * r2[:, :, 512 * e:512 * e + 512]
    o2 = jnp.maximum(acc, 0.0)
    feat = _mm(jnp.sum(o2, axis=1), ph_ref[...])
    out_ref[...] = _mm(feat, fcw_ref[...]) + fcb_ref[...]


def _net(xw, stats, bnsc, bnbi, wbs, wb0, wb1, wb2,
         r0w1, r0b1, r0w2, r0b2, r1w1, r1b1, r1w2, r1b2, fcw, fcb, bs):
    return pl.pallas_call(
        _net_body,
        grid=(_B // bs,),
        in_specs=[
            pl.BlockSpec((bs, 3, 32, 32), lambda i: (i, 0, 0, 0)),
            pl.BlockSpec((8, 16), lambda i: (0, 0)),
            pl.BlockSpec((1, 16), lambda i: (0, 0)),
            pl.BlockSpec((1, 16), lambda i: (0, 0)),
            pl.BlockSpec((288, 512), lambda i: (0, 0)),
            pl.BlockSpec((1536, 512), lambda i: (0, 0)),
            pl.BlockSpec((1536, 1024), lambda i: (0, 0)),
            pl.BlockSpec((1536, 2048), lambda i: (0, 0)),
            pl.BlockSpec((1024, 128), lambda i: (0, 0)),
            pl.BlockSpec((1, 128), lambda i: (0, 0)),
            pl.BlockSpec((128, 2), lambda i: (0, 0)),
            pl.BlockSpec((1, 2), lambda i: (0, 0)),
            pl.BlockSpec((512, 128), lambda i: (0, 0)),
            pl.BlockSpec((1, 128), lambda i: (0, 0)),
            pl.BlockSpec((128, 4), lambda i: (0, 0)),
            pl.BlockSpec((1, 4), lambda i: (0, 0)),
            pl.BlockSpec((64, 10), lambda i: (0, 0)),
            pl.BlockSpec((1, 10), lambda i: (0, 0)),
            pl.BlockSpec((16, 512), lambda i: (0, 0)),
            pl.BlockSpec((512, 128), lambda i: (0, 0)),
            pl.BlockSpec((512, 128), lambda i: (0, 0)),
            pl.BlockSpec((512, 64), lambda i: (0, 0)),
        ],
        out_specs=pl.BlockSpec((bs, 10), lambda i: (i, 0)),
        out_shape=jax.ShapeDtypeStruct((_B, 10), jnp.float32),
    )(xw, stats, bnsc, bnbi, wbs, wb0, wb1, wb2,
      r0w1, r0b1, r0w2, r0b2, r1w1, r1b1, r1w2, r1b2, fcw, fcb,
      jnp.asarray(_RT16), jnp.asarray(_P0), jnp.asarray(_P1), jnp.asarray(_PH))


def kernel(x, labels, temperature, conv1_w, bn1_scale, bn1_bias, block0_w,
           block1_ws, block2_ws, r0_w1, r0_b1, r0_w2, r0_b2,
           r1_w1, r1_b1, r1_w2, r1_b2, fc_w, fc_b):
    d1 = jnp.asarray(_D1_32)
    d2 = jnp.asarray(_D2_32)
    d2b = jnp.asarray(_D2_16)
    w1t = jnp.transpose(conv1_w, (2, 3, 1, 0))
    wbs = jnp.einsum('axo,yacp->ycxop', d1, w1t, precision=lax.Precision.HIGHEST).reshape(288, 512)
    w0t = jnp.transpose(block0_w, (2, 3, 1, 0))
    wb0 = jnp.einsum('axo,yacp->yxcop', d1, w0t, precision=lax.Precision.HIGHEST).reshape(1536, 512)
    w1e = jnp.transpose(block1_ws, (3, 4, 2, 0, 1))
    wb1 = jnp.einsum('axo,yacep->yxceop', d2, w1e, precision=lax.Precision.HIGHEST).reshape(1536, 1024)
    w2e = jnp.transpose(block2_ws, (3, 4, 2, 0, 1))
    wb2 = jnp.einsum('axo,yacep->yxceop', d2b, w2e, precision=lax.Precision.HIGHEST).reshape(1536, 2048)
    r0w1p = jnp.transpose(r0_w1.reshape(16, 8, 8, 128),
                          (1, 2, 0, 3)).reshape(1024, 128)
    r1w1p = jnp.transpose(r1_w1.reshape(32, 4, 4, 128),
                          (1, 2, 0, 3)).reshape(512, 128)

    stats = _stats(x, wbs, bs=32)
    logits = _net(x, stats, bn1_scale.reshape(1, 16), bn1_bias.reshape(1, 16),
                  wbs, wb0, wb1, wb2,
                  r0w1p, r0_b1.reshape(1, 128), r0_w2, r0_b2.reshape(1, 2),
                  r1w1p, r1_b1.reshape(1, 128), r1_w2, r1_b2.reshape(1, 4),
                  fc_w, fc_b.reshape(1, 10), bs=32)
    return logits


# speedup vs baseline: 2.6962x; 1.1975x over previous
import numpy as np

import jax
import jax.numpy as jnp
from jax import lax
from jax.experimental import pallas as pl
from jax.experimental.pallas import tpu as pltpu

_B = 512
_EPS = 1e-5


def _band1(w_in):
    d = np.zeros((3, w_in, w_in), np.float32)
    for dx in range(3):
        for xo in range(w_in):
            xi = xo + dx - 1
            if 0 <= xi < w_in:
                d[dx, xi, xo] = 1.0
    return d


def _band2(w_in):
    w_out = w_in // 2
    d = np.zeros((3, w_in, w_out), np.float32)
    for dx in range(3):
        for xo in range(w_out):
            xi = 2 * xo + dx
            if xi < w_in:
                d[dx, xi, xo] = 1.0
    return d


def _pool_mat(w_in, c, k, scale):
    p = np.zeros((w_in * c, (w_in // k) * c), np.float32)
    for x in range(w_in):
        for ch in range(c):
            p[x * c + ch, (x // k) * c + ch] = scale
    return p


def _chan_fold(w_in, c):
    r = np.zeros((w_in * c, c), np.float32)
    for x in range(w_in):
        for ch in range(c):
            r[x * c + ch, ch] = 1.0
    return r


_D1_32 = _band1(32)
_D2_32 = _band2(32)
_D2_16 = _band2(16)
_P0 = _pool_mat(32, 16, 4, 1.0 / 16.0)
_P1 = _pool_mat(16, 32, 4, 1.0 / 16.0)
_PH = _chan_fold(8, 64) / 64.0
_R16 = _chan_fold(32, 16)
_RT16 = _chan_fold(32, 16).T


def _rowshift(v, s):
    bs, h, l = v.shape
    z = jnp.zeros((bs, 1, l), jnp.float32)
    if s == -1:
        return jnp.concatenate([z, v[:, :h - 1]], axis=1)
    if s == 1:
        return jnp.concatenate([v[:, 1:], z], axis=1)
    return v


def _im_s1(v):
    bs, h, l = v.shape
    im = jnp.concatenate([_rowshift(v, dy - 1) for dy in range(3)], axis=-1)
    return im.reshape(bs * h, 3 * l)


def _im_stem(x_ref):
    return _im_s1(x_ref[...])


def _im_s2(v):
    bs, h2, l = v.shape
    h = h2 // 2
    par = v.reshape(bs, h, 2, l)
    ev = par[:, :, 0]
    od = par[:, :, 1]
    z = jnp.zeros((bs, 1, l), jnp.float32)
    ev1 = jnp.concatenate([ev[:, 1:], z], axis=1)
    im = jnp.concatenate([ev, od, ev1], axis=-1)
    return im.reshape(bs * h, 3 * l)


def _mm(a, b):
    return jnp.dot(a, b, preferred_element_type=jnp.float32)


def _mmx(a, b):
    return jnp.dot(a, b, preferred_element_type=jnp.float32,
                   precision=lax.Precision.HIGHEST)


def _stats_body(x_ref, wbs_ref, r16_ref, stats_ref):
    bs = x_ref.shape[0]
    y = _mm(_im_stem(x_ref), wbs_ref[...])
    r16 = r16_ref[...]
    s = _mm(jnp.sum(y, axis=0)[None, :], r16)
    s2 = _mm(jnp.sum(y * y, axis=0)[None, :], r16)
    rows = lax.broadcasted_iota(jnp.int32, (8, 16), 0)
    upd = jnp.where(rows == 0, s, jnp.where(rows == 1, s2, 0.0))
    prev = jnp.where(pl.program_id(0) == 0, 0.0, stats_ref[...])
    stats_ref[...] = prev + upd


def _stats(xw, wbs, bs):
    return pl.pallas_call(
        _stats_body,
        grid=(_B // bs,),
        in_specs=[
            pl.BlockSpec((bs, 32, 96), lambda i: (i, 0, 0)),
            pl.BlockSpec((288, 512), lambda i: (0, 0)),
            pl.BlockSpec((512, 16), lambda i: (0, 0)),
        ],
        out_specs=pl.BlockSpec((8, 16), lambda i: (0, 0)),
        out_shape=jax.ShapeDtypeStruct((8, 16), jnp.float32),
        compiler_params=pltpu.CompilerParams(
            dimension_semantics=("arbitrary",)),
    )(xw, wbs, jnp.asarray(_R16))


def _net_body(x_ref, stats_ref, sc_ref, bi_ref, wbs_ref, wb0_ref, wb1_ref,
              wb2_ref, r0w1_ref, r0b1_ref, r0w2_ref, r0b2_ref,
              r1w1_ref, r1b1_ref, r1w2_ref, r1b2_ref, fcw_ref, fcb_ref,
              rt16_ref, p0_ref, p1_ref, ph_ref,
              out_ref):
    bs = x_ref.shape[0]
    y = _mm(_im_stem(x_ref), wbs_ref[...])
    n = jnp.float32(_B * 1024)
    st = stats_ref[...]
    mean = st[0:1, :] / n
    var = st[1:2, :] / n - mean * mean
    inv = sc_ref[...] * lax.rsqrt(var + _EPS)
    sh = bi_ref[...] - mean * inv
    rt16 = rt16_ref[...]
    inv512 = _mm(inv, rt16)
    sh512 = _mm(sh, rt16)
    xn = jnp.maximum(y * inv512 + sh512, 0.0).reshape(bs, 32, 512)
    b0 = jnp.maximum(_mm(_im_s1(xn), wb0_ref[...]), 0.0)
    b0 = b0.reshape(bs, 32, 512)
    t = b0.reshape(bs, 8, 4, 512)
    ys = t[:, :, 0] + t[:, :, 1] + t[:, :, 2] + t[:, :, 3]
    pool0 = _mmx(ys.reshape(bs * 8, 512), p0_ref[...])
    pool0 = pool0.reshape(bs, 8, 128).reshape(bs, 1024)
    h0 = jnp.maximum(_mmx(pool0, r0w1_ref[...]) + r0b1_ref[...], 0.0)
    lg0 = _mmx(h0, r0w2_ref[...]) + r0b2_ref[...]
    m0 = (lg0[:, 1:2] > lg0[:, 0:1])[:, :, None]
    r1 = _mm(_im_s2(b0), wb1_ref[...]).reshape(bs, 16, 1024)
    o1 = jnp.maximum(jnp.where(m0, r1[:, :, 512:], r1[:, :, :512]), 0.0)
    t1 = o1.reshape(bs, 4, 4, 512)
    ys1 = t1[:, :, 0] + t1[:, :, 1] + t1[:, :, 2] + t1[:, :, 3]
    pool1 = _mmx(ys1.reshape(bs * 4, 512), p1_ref[...])
    pool1 = pool1.reshape(bs, 4, 128).reshape(bs, 512)
    h1 = jnp.maximum(_mmx(pool1, r1w1_ref[...]) + r1b1_ref[...], 0.0)
    lg1 = _mmx(h1, r1w2_ref[...]) + r1b2_ref[...]
    mx = jnp.max(lg1, axis=1, keepdims=True)
    colid = lax.broadcasted_iota(jnp.int32, lg1.shape, 1)
    idx1 = jnp.min(jnp.where(lg1 == mx, colid, 4), axis=1,
                   keepdims=True)
    r2 = _mm(_im_s2(o1), wb2_ref[...]).reshape(bs, 8, 2048)
    acc = jnp.zeros((bs, 8, 512), jnp.float32)
    for e in range(4):
        me = (idx1 == e).astype(jnp.float32)[:, :, None]
        acc = acc + me * r2[:, :, 512 * e:512 * e + 512]
    o2 = jnp.maximum(acc, 0.0)
    feat = _mm(jnp.sum(o2, axis=1), ph_ref[...])
    out_ref[...] = _mm(feat, fcw_ref[...]) + fcb_ref[...]


def _net(xw, stats, bnsc, bnbi, wbs, wb0, wb1, wb2,
         r0w1, r0b1, r0w2, r0b2, r1w1, r1b1, r1w2, r1b2, fcw, fcb, bs):
    return pl.pallas_call(
        _net_body,
        grid=(_B // bs,),
        in_specs=[
            pl.BlockSpec((bs, 32, 96), lambda i: (i, 0, 0)),
            pl.BlockSpec((8, 16), lambda i: (0, 0)),
            pl.BlockSpec((1, 16), lambda i: (0, 0)),
            pl.BlockSpec((1, 16), lambda i: (0, 0)),
            pl.BlockSpec((288, 512), lambda i: (0, 0)),
            pl.BlockSpec((1536, 512), lambda i: (0, 0)),
            pl.BlockSpec((1536, 1024), lambda i: (0, 0)),
            pl.BlockSpec((1536, 2048), lambda i: (0, 0)),
            pl.BlockSpec((1024, 128), lambda i: (0, 0)),
            pl.BlockSpec((1, 128), lambda i: (0, 0)),
            pl.BlockSpec((128, 2), lambda i: (0, 0)),
            pl.BlockSpec((1, 2), lambda i: (0, 0)),
            pl.BlockSpec((512, 128), lambda i: (0, 0)),
            pl.BlockSpec((1, 128), lambda i: (0, 0)),
            pl.BlockSpec((128, 4), lambda i: (0, 0)),
            pl.BlockSpec((1, 4), lambda i: (0, 0)),
            pl.BlockSpec((64, 10), lambda i: (0, 0)),
            pl.BlockSpec((1, 10), lambda i: (0, 0)),
            pl.BlockSpec((16, 512), lambda i: (0, 0)),
            pl.BlockSpec((512, 128), lambda i: (0, 0)),
            pl.BlockSpec((512, 128), lambda i: (0, 0)),
            pl.BlockSpec((512, 64), lambda i: (0, 0)),
        ],
        out_specs=pl.BlockSpec((bs, 10), lambda i: (i, 0)),
        out_shape=jax.ShapeDtypeStruct((_B, 10), jnp.float32),
    )(xw, stats, bnsc, bnbi, wbs, wb0, wb1, wb2,
      r0w1, r0b1, r0w2, r0b2, r1w1, r1b1, r1w2, r1b2, fcw, fcb,
      jnp.asarray(_RT16), jnp.asarray(_P0), jnp.asarray(_P1), jnp.asarray(_PH))


def kernel(x, labels, temperature, conv1_w, bn1_scale, bn1_bias, block0_w,
           block1_ws, block2_ws, r0_w1, r0_b1, r0_w2, r0_b2,
           r1_w1, r1_b1, r1_w2, r1_b2, fc_w, fc_b):
    d1 = jnp.asarray(_D1_32)
    d2 = jnp.asarray(_D2_32)
    d2b = jnp.asarray(_D2_16)
    w1t = jnp.transpose(conv1_w, (2, 3, 1, 0))
    wbs = jnp.einsum('axo,yacp->ycxop', d1, w1t, precision=lax.Precision.HIGHEST).reshape(288, 512)
    w0t = jnp.transpose(block0_w, (2, 3, 1, 0))
    wb0 = jnp.einsum('axo,yacp->yxcop', d1, w0t, precision=lax.Precision.HIGHEST).reshape(1536, 512)
    w1e = jnp.transpose(block1_ws, (3, 4, 2, 0, 1))
    wb1 = jnp.einsum('axo,yacep->yxceop', d2, w1e, precision=lax.Precision.HIGHEST).reshape(1536, 1024)
    w2e = jnp.transpose(block2_ws, (3, 4, 2, 0, 1))
    wb2 = jnp.einsum('axo,yacep->yxceop', d2b, w2e, precision=lax.Precision.HIGHEST).reshape(1536, 2048)
    r0w1p = jnp.transpose(r0_w1.reshape(16, 8, 8, 128),
                          (1, 2, 0, 3)).reshape(1024, 128)
    r1w1p = jnp.transpose(r1_w1.reshape(32, 4, 4, 128),
                          (1, 2, 0, 3)).reshape(512, 128)

    xw = jnp.transpose(x, (0, 2, 1, 3)).reshape(_B, 32, 96)
    stats = _stats(xw, wbs, bs=32)
    logits = _net(xw, stats, bn1_scale.reshape(1, 16), bn1_bias.reshape(1, 16),
                  wbs, wb0, wb1, wb2,
                  r0w1p, r0_b1.reshape(1, 128), r0_w2, r0_b2.reshape(1, 2),
                  r1w1p, r1_b1.reshape(1, 128), r1_w2, r1_b2.reshape(1, 4),
                  fc_w, fc_b.reshape(1, 10), bs=32)
    return logits


# speedup vs baseline: 2.7761x; 1.0296x over previous
import numpy as np

import jax
import jax.numpy as jnp
from jax import lax
from jax.experimental import pallas as pl
from jax.experimental.pallas import tpu as pltpu

_B = 512
_EPS = 1e-5


def _band1(w_in):
    d = np.zeros((3, w_in, w_in), np.float32)
    for dx in range(3):
        for xo in range(w_in):
            xi = xo + dx - 1
            if 0 <= xi < w_in:
                d[dx, xi, xo] = 1.0
    return d


def _band2(w_in):
    w_out = w_in // 2
    d = np.zeros((3, w_in, w_out), np.float32)
    for dx in range(3):
        for xo in range(w_out):
            xi = 2 * xo + dx
            if xi < w_in:
                d[dx, xi, xo] = 1.0
    return d


def _pool_mat(w_in, c, k, scale):
    p = np.zeros((w_in * c, (w_in // k) * c), np.float32)
    for x in range(w_in):
        for ch in range(c):
            p[x * c + ch, (x // k) * c + ch] = scale
    return p


def _chan_fold(w_in, c):
    r = np.zeros((w_in * c, c), np.float32)
    for x in range(w_in):
        for ch in range(c):
            r[x * c + ch, ch] = 1.0
    return r


_D1_32 = _band1(32)
_D2_32 = _band2(32)
_D2_16 = _band2(16)
_P0 = _pool_mat(32, 16, 4, 1.0 / 16.0)
_P1 = _pool_mat(16, 32, 4, 1.0 / 16.0)
_PH = _chan_fold(8, 64) / 64.0
_R16 = _chan_fold(32, 16)
_RT16 = _chan_fold(32, 16).T


def _rowshift(v, s):
    bs, h, l = v.shape
    z = jnp.zeros((bs, 1, l), jnp.float32)
    if s == -1:
        return jnp.concatenate([z, v[:, :h - 1]], axis=1)
    if s == 1:
        return jnp.concatenate([v[:, 1:], z], axis=1)
    return v


def _im_s1(v):
    bs, h, l = v.shape
    im = jnp.concatenate([_rowshift(v, dy - 1) for dy in range(3)], axis=-1)
    return im.reshape(bs * h, 3 * l)


def _im_stem(x_ref):
    return _im_s1(x_ref[...])


def _im_s2(v):
    bs, h2, l = v.shape
    h = h2 // 2
    par = v.reshape(bs, h, 2, l)
    ev = par[:, :, 0]
    od = par[:, :, 1]
    z = jnp.zeros((bs, 1, l), jnp.float32)
    ev1 = jnp.concatenate([ev[:, 1:], z], axis=1)
    im = jnp.concatenate([ev, od, ev1], axis=-1)
    return im.reshape(bs * h, 3 * l)


def _mm(a, b):
    return jnp.dot(a, b, preferred_element_type=jnp.float32)


def _mmx(a, b):
    return jnp.dot(a, b, preferred_element_type=jnp.float32,
                   precision=lax.Precision.HIGHEST)


def _stats_body(x_ref, wbs_ref, r16_ref, stats_ref):
    bs = x_ref.shape[0]
    y = _mm(_im_stem(x_ref), wbs_ref[...])
    r16 = r16_ref[...]
    s = _mm(jnp.sum(y, axis=0)[None, :], r16)
    s2 = _mm(jnp.sum(y * y, axis=0)[None, :], r16)
    rows = lax.broadcasted_iota(jnp.int32, (8, 16), 0)
    upd = jnp.where(rows == 0, s, jnp.where(rows == 1, s2, 0.0))
    prev = jnp.where(pl.program_id(0) == 0, 0.0, stats_ref[...])
    stats_ref[...] = prev + upd


def _stats(xw, wbs, bs):
    return pl.pallas_call(
        _stats_body,
        grid=(_B // bs,),
        in_specs=[
            pl.BlockSpec((bs, 32, 96), lambda i: (i, 0, 0)),
            pl.BlockSpec((288, 512), lambda i: (0, 0)),
            pl.BlockSpec((512, 16), lambda i: (0, 0)),
        ],
        out_specs=pl.BlockSpec((8, 16), lambda i: (0, 0)),
        out_shape=jax.ShapeDtypeStruct((8, 16), jnp.float32),
        compiler_params=pltpu.CompilerParams(
            dimension_semantics=("arbitrary",)),
    )(xw, wbs, jnp.asarray(_R16))


def _net_body(x_ref, stats_ref, sc_ref, bi_ref, wbs_ref, wb0_ref, wb1_ref,
              wb2_ref, r0w1_ref, r0b1_ref, r0w2_ref, r0b2_ref,
              r1w1_ref, r1b1_ref, r1w2_ref, r1b2_ref, fcw_ref, fcb_ref,
              rt16_ref, p0_ref, p1_ref, ph_ref,
              out_ref):
    bs = x_ref.shape[0]
    y = _mm(_im_stem(x_ref), wbs_ref[...])
    n = jnp.float32(_B * 1024)
    st = stats_ref[...]
    mean = st[0:1, :] / n
    var = st[1:2, :] / n - mean * mean
    inv = sc_ref[...] * lax.rsqrt(var + _EPS)
    sh = bi_ref[...] - mean * inv
    rt16 = rt16_ref[...]
    inv512 = _mm(inv, rt16)
    sh512 = _mm(sh, rt16)
    xn = jnp.maximum(y * inv512 + sh512, 0.0).reshape(bs, 32, 512)
    b0 = jnp.maximum(_mm(_im_s1(xn), wb0_ref[...]), 0.0)
    b0 = b0.reshape(bs, 32, 512)
    t = b0.reshape(bs, 8, 4, 512)
    ys = t[:, :, 0] + t[:, :, 1] + t[:, :, 2] + t[:, :, 3]
    pool0 = _mmx(ys.reshape(bs * 8, 512), p0_ref[...])
    pool0 = pool0.reshape(bs, 8, 128).reshape(bs, 1024)
    h0 = jnp.maximum(_mmx(pool0, r0w1_ref[...]) + r0b1_ref[...], 0.0)
    lg0 = _mmx(h0, r0w2_ref[...]) + r0b2_ref[...]
    m0 = (lg0[:, 1:2] > lg0[:, 0:1])[:, :, None]
    r1 = _mm(_im_s2(b0), wb1_ref[...]).reshape(bs, 16, 1024)
    o1 = jnp.maximum(jnp.where(m0, r1[:, :, 512:], r1[:, :, :512]), 0.0)
    t1 = o1.reshape(bs, 4, 4, 512)
    ys1 = t1[:, :, 0] + t1[:, :, 1] + t1[:, :, 2] + t1[:, :, 3]
    pool1 = _mmx(ys1.reshape(bs * 4, 512), p1_ref[...])
    pool1 = pool1.reshape(bs, 4, 128).reshape(bs, 512)
    h1 = jnp.maximum(_mmx(pool1, r1w1_ref[...]) + r1b1_ref[...], 0.0)
    lg1 = _mmx(h1, r1w2_ref[...]) + r1b2_ref[...]
    mx = jnp.max(lg1, axis=1, keepdims=True)
    colid = lax.broadcasted_iota(jnp.int32, lg1.shape, 1)
    idx1 = jnp.min(jnp.where(lg1 == mx, colid, 4), axis=1,
                   keepdims=True)
    r2 = _mm(_im_s2(o1), wb2_ref[...]).reshape(bs, 8, 2048)
    acc = jnp.zeros((bs, 8, 512), jnp.float32)
    for e in range(4):
        me = (idx1 == e).astype(jnp.float32)[:, :, None]
        acc = acc + me * r2[:, :, 512 * e:512 * e + 512]
    o2 = jnp.maximum(acc, 0.0)
    feat = _mm(jnp.sum(o2, axis=1), ph_ref[...])
    out_ref[...] = _mm(feat, fcw_ref[...]) + fcb_ref[...]


def _net(xw, stats, bnsc, bnbi, wbs, wb0, wb1, wb2,
         r0w1, r0b1, r0w2, r0b2, r1w1, r1b1, r1w2, r1b2, fcw, fcb, bs):
    return pl.pallas_call(
        _net_body,
        grid=(_B // bs,),
        in_specs=[
            pl.BlockSpec((bs, 32, 96), lambda i: (i, 0, 0)),
            pl.BlockSpec((8, 16), lambda i: (0, 0)),
            pl.BlockSpec((1, 16), lambda i: (0, 0)),
            pl.BlockSpec((1, 16), lambda i: (0, 0)),
            pl.BlockSpec((288, 512), lambda i: (0, 0)),
            pl.BlockSpec((1536, 512), lambda i: (0, 0)),
            pl.BlockSpec((1536, 1024), lambda i: (0, 0)),
            pl.BlockSpec((1536, 2048), lambda i: (0, 0)),
            pl.BlockSpec((1024, 128), lambda i: (0, 0)),
            pl.BlockSpec((1, 128), lambda i: (0, 0)),
            pl.BlockSpec((128, 2), lambda i: (0, 0)),
            pl.BlockSpec((1, 2), lambda i: (0, 0)),
            pl.BlockSpec((512, 128), lambda i: (0, 0)),
            pl.BlockSpec((1, 128), lambda i: (0, 0)),
            pl.BlockSpec((128, 4), lambda i: (0, 0)),
            pl.BlockSpec((1, 4), lambda i: (0, 0)),
            pl.BlockSpec((64, 10), lambda i: (0, 0)),
            pl.BlockSpec((1, 10), lambda i: (0, 0)),
            pl.BlockSpec((16, 512), lambda i: (0, 0)),
            pl.BlockSpec((512, 128), lambda i: (0, 0)),
            pl.BlockSpec((512, 128), lambda i: (0, 0)),
            pl.BlockSpec((512, 64), lambda i: (0, 0)),
        ],
        out_specs=pl.BlockSpec((bs, 10), lambda i: (i, 0)),
        out_shape=jax.ShapeDtypeStruct((_B, 10), jnp.float32),
    )(xw, stats, bnsc, bnbi, wbs, wb0, wb1, wb2,
      r0w1, r0b1, r0w2, r0b2, r1w1, r1b1, r1w2, r1b2, fcw, fcb,
      jnp.asarray(_RT16), jnp.asarray(_P0), jnp.asarray(_P1), jnp.asarray(_PH))


def kernel(x, labels, temperature, conv1_w, bn1_scale, bn1_bias, block0_w,
           block1_ws, block2_ws, r0_w1, r0_b1, r0_w2, r0_b2,
           r1_w1, r1_b1, r1_w2, r1_b2, fc_w, fc_b):
    d1 = jnp.asarray(_D1_32)
    d2 = jnp.asarray(_D2_32)
    d2b = jnp.asarray(_D2_16)

    def bw_xc(dmat, w):
        xi, xo = dmat.shape
        ci, co = w.shape
        return (dmat[:, None, :, None] * w[None, :, None, :]) \
            .reshape(xi * ci, xo * co)

    def bw_cx(dmat, w):
        xi, xo = dmat.shape
        ci, co = w.shape
        return (w[:, None, None, :] * dmat[None, :, :, None]) \
            .reshape(ci * xi, xo * co)

    w1t = jnp.transpose(conv1_w, (2, 3, 1, 0))
    wbs = jnp.concatenate(
        [sum(bw_cx(d1[dx], w1t[dy, dx]) for dx in range(3))
         for dy in range(3)], axis=0)
    w0t = jnp.transpose(block0_w, (2, 3, 1, 0))
    wb0 = jnp.concatenate(
        [sum(bw_xc(d1[dx], w0t[dy, dx]) for dx in range(3))
         for dy in range(3)], axis=0)
    w1e = jnp.transpose(block1_ws, (3, 4, 2, 0, 1))
    wb1 = jnp.concatenate(
        [jnp.concatenate(
            [sum(bw_xc(d2[dx], w1e[dy, dx, :, e]) for dx in range(3))
             for e in range(2)], axis=1)
         for dy in range(3)], axis=0)
    w2e = jnp.transpose(block2_ws, (3, 4, 2, 0, 1))
    wb2 = jnp.concatenate(
        [jnp.concatenate(
            [sum(bw_xc(d2b[dx], w2e[dy, dx, :, e]) for dx in range(3))
             for e in range(4)], axis=1)
         for dy in range(3)], axis=0)
    r0w1p = jnp.transpose(r0_w1.reshape(16, 8, 8, 128),
                          (1, 2, 0, 3)).reshape(1024, 128)
    r1w1p = jnp.transpose(r1_w1.reshape(32, 4, 4, 128),
                          (1, 2, 0, 3)).reshape(512, 128)

    xw = jnp.transpose(x, (0, 2, 1, 3)).reshape(_B, 32, 96)
    stats = _stats(xw, wbs, bs=32)
    logits = _net(xw, stats, bn1_scale.reshape(1, 16), bn1_bias.reshape(1, 16),
                  wbs, wb0, wb1, wb2,
                  r0w1p, r0_b1.reshape(1, 128), r0_w2, r0_b2.reshape(1, 2),
                  r1w1p, r1_b1.reshape(1, 128), r1_w2, r1_b2.reshape(1, 4),
                  fc_w, fc_b.reshape(1, 10), bs=32)
    return logits
